# baseline (device time: 60117 ns/iter reference)
import jax
import jax.numpy as jnp
from jax import lax
from jax.experimental import pallas as pl
from jax.experimental.pallas import tpu as pltpu

N_DEV = 8


def kernel(x, w_mat):
    k_total, k_local = x.shape
    _, n = w_mat.shape
    chunk = k_total // N_DEV

    def body(x_ref, w_ref, out_ref, acc_ref, recv_ref, send_sems, recv_sems):
        my = lax.axis_index("i")
        left = lax.rem(my + (N_DEV - 1), N_DEV)
        right = lax.rem(my + 1, N_DEV)

        barrier_sem = pltpu.get_barrier_semaphore()
        for nbr in (left, right):
            pl.semaphore_signal(
                barrier_sem, inc=1,
                device_id=(nbr,), device_id_type=pl.DeviceIdType.MESH,
            )
        pl.semaphore_wait(barrier_sem, 2)

        def partial_chunk(c):
            rows = x_ref[pl.ds(c * chunk, chunk), :]
            return jnp.dot(rows, w_ref[:, :], preferred_element_type=jnp.float32)

        acc_ref[:, :] = partial_chunk(lax.rem(my + (N_DEV - 1), N_DEV))

        for s in range(N_DEV - 1):
            rdma = pltpu.make_async_remote_copy(
                src_ref=acc_ref,
                dst_ref=recv_ref.at[s],
                send_sem=send_sems.at[s],
                recv_sem=recv_sems.at[s],
                device_id=(right,),
                device_id_type=pl.DeviceIdType.MESH,
            )
            rdma.start()
            rdma.wait()
            c = lax.rem(my + (2 * N_DEV - 2 - s), N_DEV)
            if s < N_DEV - 2:
                acc_ref[:, :] = recv_ref[s] + partial_chunk(c)
            else:
                out_ref[:, :] = recv_ref[s] + partial_chunk(c)

    return pl.pallas_call(
        body,
        out_shape=jax.ShapeDtypeStruct((chunk, n), jnp.float32),
        in_specs=[
            pl.BlockSpec(memory_space=pltpu.VMEM),
            pl.BlockSpec(memory_space=pltpu.VMEM),
        ],
        out_specs=pl.BlockSpec(memory_space=pltpu.VMEM),
        scratch_shapes=[
            pltpu.VMEM((chunk, n), jnp.float32),
            pltpu.VMEM((N_DEV - 1, chunk, n), jnp.float32),
            pltpu.SemaphoreType.DMA((N_DEV - 1,)),
            pltpu.SemaphoreType.DMA((N_DEV - 1,)),
        ],
        compiler_params=pltpu.CompilerParams(collective_id=0),
    )(x, w_mat)


# device time: 38702 ns/iter; 1.5533x vs baseline; 1.5533x over previous
import jax
import jax.numpy as jnp
from jax import lax
from jax.experimental import pallas as pl
from jax.experimental.pallas import tpu as pltpu

N_DEV = 8
RING = (0, 1, 2, 3, 7, 6, 5, 4)


def kernel(x, w_mat):
    k_total, k_local = x.shape
    _, n = w_mat.shape
    chunk = k_total // N_DEV
    nh = n // 2

    def body(x_ref, w_ref, out_ref,
             acc_r, acc_l, recv_r, recv_l,
             send_sems_r, recv_sems_r, send_sems_l, recv_sems_l):
        def ring(i):
            return jnp.where(i < 4, i, 11 - i)

        my = lax.axis_index("i")
        p = ring(my)
        right = ring(lax.rem(p + 1, N_DEV))
        left = ring(lax.rem(p + (N_DEV - 1), N_DEV))

        barrier_sem = pltpu.get_barrier_semaphore()
        for nbr in (left, right):
            pl.semaphore_signal(
                barrier_sem, inc=1,
                device_id=(nbr,), device_id_type=pl.DeviceIdType.MESH,
            )
        pl.semaphore_wait(barrier_sem, 2)

        def partial_r(c):
            rows = x_ref[pl.ds(c * chunk, chunk), :]
            return jnp.dot(rows, w_ref[:, :nh], preferred_element_type=jnp.float32)

        def partial_l(c):
            rows = x_ref[pl.ds(c * chunk, chunk), :]
            return jnp.dot(rows, w_ref[:, nh:], preferred_element_type=jnp.float32)

        acc_r[:, :] = partial_r(ring(lax.rem(p + (N_DEV - 1), N_DEV)))
        acc_l[:, :] = partial_l(ring(lax.rem(p + 1, N_DEV)))

        for s in range(N_DEV - 1):
            rdma_r = pltpu.make_async_remote_copy(
                src_ref=acc_r,
                dst_ref=recv_r.at[s],
                send_sem=send_sems_r.at[s],
                recv_sem=recv_sems_r.at[s],
                device_id=(right,),
                device_id_type=pl.DeviceIdType.MESH,
            )
            rdma_l = pltpu.make_async_remote_copy(
                src_ref=acc_l,
                dst_ref=recv_l.at[s],
                send_sem=send_sems_l.at[s],
                recv_sem=recv_sems_l.at[s],
                device_id=(left,),
                device_id_type=pl.DeviceIdType.MESH,
            )
            rdma_r.start()
            rdma_l.start()
            cr = ring(lax.rem(p + (2 * N_DEV - 2 - s), N_DEV))
            cl = ring(lax.rem(p + 2 + s, N_DEV))
            pr = partial_r(cr)
            pl_ = partial_l(cl)
            rdma_r.wait()
            rdma_l.wait()
            if s < N_DEV - 2:
                acc_r[:, :] = recv_r[s] + pr
                acc_l[:, :] = recv_l[s] + pl_
            else:
                out_ref[:, :nh] = recv_r[s] + pr
                out_ref[:, nh:] = recv_l[s] + pl_

    return pl.pallas_call(
        body,
        out_shape=jax.ShapeDtypeStruct((chunk, n), jnp.float32),
        in_specs=[
            pl.BlockSpec(memory_space=pltpu.VMEM),
            pl.BlockSpec(memory_space=pltpu.VMEM),
        ],
        out_specs=pl.BlockSpec(memory_space=pltpu.VMEM),
        scratch_shapes=[
            pltpu.VMEM((chunk, nh), jnp.float32),
            pltpu.VMEM((chunk, nh), jnp.float32),
            pltpu.VMEM((N_DEV - 1, chunk, nh), jnp.float32),
            pltpu.VMEM((N_DEV - 1, chunk, nh), jnp.float32),
            pltpu.SemaphoreType.DMA((N_DEV - 1,)),
            pltpu.SemaphoreType.DMA((N_DEV - 1,)),
            pltpu.SemaphoreType.DMA((N_DEV - 1,)),
            pltpu.SemaphoreType.DMA((N_DEV - 1,)),
        ],
        compiler_params=pltpu.CompilerParams(collective_id=0),
    )(x, w_mat)


# device time: 29218 ns/iter; 2.0575x vs baseline; 1.3246x over previous
import jax
import jax.numpy as jnp
from jax import lax
from jax.experimental import pallas as pl
from jax.experimental.pallas import tpu as pltpu

N_DEV = 8
P = 4


def kernel(x, w_mat):
    k_total, k_local = x.shape
    _, n = w_mat.shape
    chunk = k_total // N_DEV
    nh = n // 2
    rp = chunk // P

    def body(x_ref, w_ref, out_ref,
             seed_r, seed_l, recv_r, recv_l,
             send_sems_r, recv_sems_r, send_sems_l, recv_sems_l):
        def ring(i):
            return jnp.where(i < 4, i, 11 - i)

        my = lax.axis_index("i")
        p = ring(my)
        right = ring(lax.rem(p + 1, N_DEV))
        left = ring(lax.rem(p + (N_DEV - 1), N_DEV))

        barrier_sem = pltpu.get_barrier_semaphore()
        for nbr in (left, right):
            pl.semaphore_signal(
                barrier_sem, inc=1,
                device_id=(nbr,), device_id_type=pl.DeviceIdType.MESH,
            )
        pl.semaphore_wait(barrier_sem, 2)

        def partial_r(c):
            rows = x_ref[pl.ds(c * chunk, chunk), :]
            return jnp.dot(rows, w_ref[:, :nh], preferred_element_type=jnp.float32)

        def partial_l(c):
            rows = x_ref[pl.ds(c * chunk, chunk), :]
            return jnp.dot(rows, w_ref[:, nh:], preferred_element_type=jnp.float32)

        def send(src, s, j, dst_slots, send_sems, recv_sems, target):
            rdma = pltpu.make_async_remote_copy(
                src_ref=src,
                dst_ref=dst_slots.at[s, pl.ds(j * rp, rp), :],
                send_sem=send_sems.at[s, j],
                recv_sem=recv_sems.at[s, j],
                device_id=(target,),
                device_id_type=pl.DeviceIdType.MESH,
            )
            rdma.start()
            return rdma

        def wait_piece(slots, s, j, recv_sems):
            ref = slots.at[s, pl.ds(j * rp, rp), :]
            pltpu.make_async_remote_copy(
                src_ref=ref, dst_ref=ref,
                send_sem=recv_sems.at[s, j], recv_sem=recv_sems.at[s, j],
                device_id=(right,), device_id_type=pl.DeviceIdType.MESH,
            ).wait_recv()

        descs = []

        seed_r[:, :] = partial_r(ring(lax.rem(p + (N_DEV - 1), N_DEV)))
        seed_l[:, :] = partial_l(ring(lax.rem(p + 1, N_DEV)))
        for j in range(P):
            descs.append(send(seed_r.at[pl.ds(j * rp, rp), :], 0, j,
                              recv_r, send_sems_r, recv_sems_r, right))
            descs.append(send(seed_l.at[pl.ds(j * rp, rp), :], 0, j,
                              recv_l, send_sems_l, recv_sems_l, left))

        for s in range(1, N_DEV - 1):
            cr = ring(lax.rem(p + (2 * N_DEV - 1 - s), N_DEV))
            cl = ring(lax.rem(p + 1 + s, N_DEV))
            pr = partial_r(cr)
            pl_ = partial_l(cl)
            for j in range(P):
                rows = pl.ds(j * rp, rp)
                wait_piece(recv_r, s - 1, j, recv_sems_r)
                recv_r[s - 1, rows, :] += pr[j * rp:(j + 1) * rp, :]
                descs.append(send(recv_r.at[s - 1, rows, :], s, j,
                                  recv_r, send_sems_r, recv_sems_r, right))
                wait_piece(recv_l, s - 1, j, recv_sems_l)
                recv_l[s - 1, rows, :] += pl_[j * rp:(j + 1) * rp, :]
                descs.append(send(recv_l.at[s - 1, rows, :], s, j,
                                  recv_l, send_sems_l, recv_sems_l, left))

        pr = partial_r(my)
        pl_ = partial_l(my)
        for j in range(P):
            rows = pl.ds(j * rp, rp)
            wait_piece(recv_r, N_DEV - 2, j, recv_sems_r)
            out_ref[rows, :nh] = recv_r[N_DEV - 2, rows, :] + pr[j * rp:(j + 1) * rp, :]
            wait_piece(recv_l, N_DEV - 2, j, recv_sems_l)
            out_ref[rows, nh:] = recv_l[N_DEV - 2, rows, :] + pl_[j * rp:(j + 1) * rp, :]

        for d in descs:
            d.wait_send()

    return pl.pallas_call(
        body,
        out_shape=jax.ShapeDtypeStruct((chunk, n), jnp.float32),
        in_specs=[
            pl.BlockSpec(memory_space=pltpu.VMEM),
            pl.BlockSpec(memory_space=pltpu.VMEM),
        ],
        out_specs=pl.BlockSpec(memory_space=pltpu.VMEM),
        scratch_shapes=[
            pltpu.VMEM((chunk, nh), jnp.float32),
            pltpu.VMEM((chunk, nh), jnp.float32),
            pltpu.VMEM((N_DEV - 1, chunk, nh), jnp.float32),
            pltpu.VMEM((N_DEV - 1, chunk, nh), jnp.float32),
            pltpu.SemaphoreType.DMA((N_DEV - 1, P)),
            pltpu.SemaphoreType.DMA((N_DEV - 1, P)),
            pltpu.SemaphoreType.DMA((N_DEV - 1, P)),
            pltpu.SemaphoreType.DMA((N_DEV - 1, P)),
        ],
        compiler_params=pltpu.CompilerParams(collective_id=0),
    )(x, w_mat)
